# baseline (speedup 1.0000x reference)
import math

import jax
import jax.numpy as jnp
from jax.experimental import pallas as pl
from jax.experimental.pallas import tpu as pltpu

_PAD = 0
_SMOOTHING = 0.1
_CONF = 1.0 - _SMOOTHING

_L = 128
_W = 2048
_NS = _W // _L


def _make_body(size, n_blocks, smooth, c1):
    def _accumulate(c, xv, t, acc_ref, gacc_ref, first):
        xs = [xv[:, s * _L:(s + 1) * _L] for s in range(_NS)]
        vals = xs
        while len(vals) > 1:
            vals = [a + b for a, b in zip(vals[::2], vals[1::2])]
        if first:
            acc_ref[...] = vals[0]
        else:
            acc_ref[...] += vals[0]
        rowsel = (t // _W) == c
        schunk = (t % _W) // _L
        g = jnp.zeros_like(xs[0]) if first else gacc_ref[...]
        for s in range(_NS):
            g = jnp.where(jnp.logical_and(rowsel, schunk == s), xs[s], g)
        gacc_ref[...] = g

    def _body(t_ref, x_ref, out_ref, acc_ref, gacc_ref, x0_ref):
        c = pl.program_id(0)
        xblk = x_ref[...]
        t = t_ref[...]

        @pl.when(c == 0)
        def _():
            _accumulate(c, xblk, t, acc_ref, gacc_ref, first=True)
            x0_ref[...] = xblk[:, 0:1]

        @pl.when(jnp.logical_and(c > 0, c < n_blocks - 1))
        def _():
            _accumulate(c, xblk, t, acc_ref, gacc_ref, first=False)

        @pl.when(c == n_blocks - 1)
        def _():
            cols = c * _W + jax.lax.broadcasted_iota(jnp.int32, (1, _W), 1)
            xv = jnp.where(cols < size, xblk, 0.0)
            _accumulate(c, xv, t, acc_ref, gacc_ref, first=False)

            lane = jax.lax.broadcasted_iota(jnp.int32, (1, _L), 1)
            eq = lane == (t % _L)
            g = jnp.sum(jnp.where(eq, gacc_ref[...], 0.0), axis=1, keepdims=True)
            s = jnp.sum(acc_ref[...], axis=1, keepdims=True)
            mask = (t != _PAD).astype(jnp.float32)
            contrib = c1 - smooth * (s - x0_ref[...] - g) - _CONF * g
            out_ref[0, 0] = jnp.sum(mask * contrib)

    return _body


def kernel(x, target):
    b, size = x.shape
    n_blocks = (size + _W - 1) // _W
    smooth = _SMOOTHING / (size - 2)
    c1 = (size - 2) * smooth * math.log(smooth) + _CONF * math.log(_CONF)
    t2 = target.astype(jnp.int32).reshape(b, 1)
    out = pl.pallas_call(
        _make_body(size, n_blocks, smooth, c1),
        grid=(n_blocks,),
        in_specs=[
            pl.BlockSpec((b, 1), lambda c: (0, 0)),
            pl.BlockSpec((b, _W), lambda c: (0, c)),
        ],
        out_specs=pl.BlockSpec((1, 1), lambda c: (0, 0), memory_space=pltpu.SMEM),
        out_shape=jax.ShapeDtypeStruct((1, 1), jnp.float32),
        scratch_shapes=[
            pltpu.VMEM((b, _L), jnp.float32),
            pltpu.VMEM((b, _L), jnp.float32),
            pltpu.VMEM((b, 1), jnp.float32),
        ],
        compiler_params=pltpu.CompilerParams(
            dimension_semantics=("arbitrary",),
        ),
    )(t2, x)
    return out[0, 0]


# speedup vs baseline: 1.2266x; 1.2266x over previous
import functools
import math

import jax
import jax.numpy as jnp
from jax import lax
from jax.experimental import pallas as pl
from jax.experimental.pallas import tpu as pltpu
from jax.experimental.pallas import tpu_sc as plsc

_PAD = 0
_SMOOTHING = 0.1
_CONF = 1.0 - _SMOOTHING

_L = 128
_W = 4096
_NS = _W // _L

_SC_CORES = 2
_SC_SUBCORES = 16
_NW = _SC_CORES * _SC_SUBCORES


def _sc_gather(x, t32):
    b, _ = x.shape
    rpw = b // _NW
    mesh = plsc.VectorSubcoreMesh(core_axis_name="c", subcore_axis_name="s")

    @functools.partial(
        pl.kernel,
        mesh=mesh,
        out_type=[
            jax.ShapeDtypeStruct((b,), jnp.float32),
            jax.ShapeDtypeStruct((b,), jnp.float32),
        ],
        scratch_types=[
            pltpu.VMEM((rpw,), jnp.int32),
            pltpu.VMEM((rpw * 8, 128), jnp.float32),
            pltpu.VMEM((rpw, 128), jnp.float32),
            pltpu.VMEM((rpw,), jnp.float32),
            pltpu.VMEM((rpw,), jnp.float32),
            pltpu.SemaphoreType.DMA,
        ],
        compiler_params=pltpu.CompilerParams(needs_layout_passes=False),
    )
    def sc_kernel(x_hbm, t_hbm, g_hbm, x0_hbm, tbuf, tiles, x0chunk, gout, x0out, sem):
        wid = lax.axis_index("s") * _SC_CORES + lax.axis_index("c")
        base = wid * rpw
        pltpu.sync_copy(t_hbm.at[pl.ds(base, rpw)], tbuf)
        x0cp = pltpu.async_copy(
            x_hbm.at[pl.ds(base, rpw), pl.ds(0, 128)], x0chunk, sem
        )
        copies = []
        for h in range(rpw // 16):
            startv = (tbuf[pl.ds(h * 16, 16)] >> 7) << 7
            for jj in range(16):
                j = h * 16 + jj
                copies.append(pltpu.async_copy(
                    x_hbm.at[pl.ds(base + (j // 8) * 8, 8),
                             pl.ds(pl.multiple_of(startv[jj], 128), 128)],
                    tiles.at[pl.ds(j * 8, 8)],
                    sem,
                ))
        x0cp.wait()
        for cp in copies:
            cp.wait()
        iota = lax.iota(jnp.int32, 16)
        for h in range(rpw // 16):
            jvec = h * 16 + iota
            tvec = tbuf[pl.ds(h * 16, 16)]
            rowidx = jvec * 8 + lax.bitwise_and(base + jvec, 7)
            lanes = lax.bitwise_and(tvec, 127)
            gout[pl.ds(h * 16, 16)] = plsc.load_gather(tiles, [rowidx, lanes])
            x0out[pl.ds(h * 16, 16)] = plsc.load_gather(x0chunk, [jvec, iota * 0])
        pltpu.sync_copy(gout, g_hbm.at[pl.ds(base, rpw)])
        pltpu.sync_copy(x0out, x0_hbm.at[pl.ds(base, rpw)])

    return sc_kernel(x, t32)


def _make_tc_body(size, n_blocks, smooth, c1):
    def _tree_sum(xv):
        vals = [xv[:, s * _L:(s + 1) * _L] for s in range(_NS)]
        while len(vals) > 1:
            vals = [a + b for a, b in zip(vals[::2], vals[1::2])]
        return vals[0]

    def _body(t_ref, g_ref, x0_ref, x_ref, out_ref, acc_ref):
        c = pl.program_id(0)
        xblk = x_ref[...]

        @pl.when(c == 0)
        def _():
            acc_ref[...] = _tree_sum(xblk)

        @pl.when(jnp.logical_and(c > 0, c < n_blocks - 1))
        def _():
            acc_ref[...] += _tree_sum(xblk)

        @pl.when(c == n_blocks - 1)
        def _():
            cols = c * _W + lax.broadcasted_iota(jnp.int32, (1, _W), 1)
            acc_ref[...] += _tree_sum(jnp.where(cols < size, xblk, 0.0))

            t = t_ref[...]
            g = g_ref[...]
            s = jnp.sum(acc_ref[...], axis=1, keepdims=True)
            mask = (t != _PAD).astype(jnp.float32)
            contrib = c1 - smooth * (s - x0_ref[...] - g) - _CONF * g
            out_ref[0, 0] = jnp.sum(mask * contrib)

    return _body


def kernel(x, target):
    b, size = x.shape
    n_blocks = (size + _W - 1) // _W
    smooth = _SMOOTHING / (size - 2)
    c1 = (size - 2) * smooth * math.log(smooth) + _CONF * math.log(_CONF)
    t32 = target.astype(jnp.int32)
    g, x0 = _sc_gather(x, t32)
    out = pl.pallas_call(
        _make_tc_body(size, n_blocks, smooth, c1),
        grid=(n_blocks,),
        in_specs=[
            pl.BlockSpec((b, 1), lambda c: (0, 0)),
            pl.BlockSpec((b, 1), lambda c: (0, 0)),
            pl.BlockSpec((b, 1), lambda c: (0, 0)),
            pl.BlockSpec((b, _W), lambda c: (0, c)),
        ],
        out_specs=pl.BlockSpec((1, 1), lambda c: (0, 0), memory_space=pltpu.SMEM),
        out_shape=jax.ShapeDtypeStruct((1, 1), jnp.float32),
        scratch_shapes=[
            pltpu.VMEM((b, _L), jnp.float32),
        ],
        compiler_params=pltpu.CompilerParams(
            dimension_semantics=("arbitrary",),
        ),
    )(t32.reshape(b, 1), g.reshape(b, 1), x0.reshape(b, 1), x)
    return out[0, 0]


# speedup vs baseline: 1.2321x; 1.0045x over previous
import functools
import math

import jax
import jax.numpy as jnp
from jax import lax
from jax.experimental import pallas as pl
from jax.experimental.pallas import tpu as pltpu
from jax.experimental.pallas import tpu_sc as plsc

_PAD = 0
_SMOOTHING = 0.1
_CONF = 1.0 - _SMOOTHING

_L = 128
_W = 2048
_NS = _W // _L

_SC_CORES = 2
_SC_SUBCORES = 16
_NW = _SC_CORES * _SC_SUBCORES


def _sc_gather(x, t32):
    b, _ = x.shape
    rpw = b // _NW
    mesh = plsc.VectorSubcoreMesh(core_axis_name="c", subcore_axis_name="s")

    @functools.partial(
        pl.kernel,
        mesh=mesh,
        out_type=[
            jax.ShapeDtypeStruct((b,), jnp.float32),
            jax.ShapeDtypeStruct((b,), jnp.float32),
        ],
        scratch_types=[
            pltpu.VMEM((rpw,), jnp.int32),
            pltpu.VMEM((rpw * 8, 128), jnp.float32),
            pltpu.VMEM((rpw, 128), jnp.float32),
            pltpu.VMEM((rpw,), jnp.float32),
            pltpu.VMEM((rpw,), jnp.float32),
            pltpu.SemaphoreType.DMA,
        ],
        compiler_params=pltpu.CompilerParams(needs_layout_passes=False),
    )
    def sc_kernel(x_hbm, t_hbm, g_hbm, x0_hbm, tbuf, tiles, x0chunk, gout, x0out, sem):
        wid = lax.axis_index("s") * _SC_CORES + lax.axis_index("c")
        base = wid * rpw
        pltpu.sync_copy(t_hbm.at[pl.ds(base, rpw)], tbuf)
        x0cp = pltpu.async_copy(
            x_hbm.at[pl.ds(base, rpw), pl.ds(0, 128)], x0chunk, sem
        )
        copies = []
        for h in range(rpw // 16):
            startv = (tbuf[pl.ds(h * 16, 16)] >> 7) << 7
            for jj in range(16):
                j = h * 16 + jj
                copies.append(pltpu.async_copy(
                    x_hbm.at[pl.ds(base + (j // 8) * 8, 8),
                             pl.ds(pl.multiple_of(startv[jj], 128), 128)],
                    tiles.at[pl.ds(j * 8, 8)],
                    sem,
                ))
        x0cp.wait()
        for cp in copies:
            cp.wait()
        iota = lax.iota(jnp.int32, 16)
        for h in range(rpw // 16):
            jvec = h * 16 + iota
            tvec = tbuf[pl.ds(h * 16, 16)]
            rowidx = jvec * 8 + lax.bitwise_and(base + jvec, 7)
            lanes = lax.bitwise_and(tvec, 127)
            gout[pl.ds(h * 16, 16)] = plsc.load_gather(tiles, [rowidx, lanes])
            x0out[pl.ds(h * 16, 16)] = plsc.load_gather(x0chunk, [jvec, iota * 0])
        pltpu.sync_copy(gout, g_hbm.at[pl.ds(base, rpw)])
        pltpu.sync_copy(x0out, x0_hbm.at[pl.ds(base, rpw)])

    return sc_kernel(x, t32)


def _make_tc_body(size, n_blocks, smooth, c1):
    def _tree_sum(xv):
        vals = [xv[:, s * _L:(s + 1) * _L] for s in range(_NS)]
        while len(vals) > 1:
            vals = [a + b for a, b in zip(vals[::2], vals[1::2])]
        return vals[0]

    def _body(t_ref, g_ref, x0_ref, x_ref, out_ref, acc_ref):
        c = pl.program_id(0)
        xblk = x_ref[...]

        @pl.when(c == 0)
        def _():
            acc_ref[...] = _tree_sum(xblk)

        @pl.when(jnp.logical_and(c > 0, c < n_blocks - 1))
        def _():
            acc_ref[...] += _tree_sum(xblk)

        @pl.when(c == n_blocks - 1)
        def _():
            cols = c * _W + lax.broadcasted_iota(jnp.int32, (1, _W), 1)
            acc_ref[...] += _tree_sum(jnp.where(cols < size, xblk, 0.0))

            t = t_ref[...]
            g = g_ref[...]
            s = jnp.sum(acc_ref[...], axis=1, keepdims=True)
            mask = (t != _PAD).astype(jnp.float32)
            contrib = c1 - smooth * (s - x0_ref[...] - g) - _CONF * g
            out_ref[0, 0] = jnp.sum(mask * contrib)

    return _body


def kernel(x, target):
    b, size = x.shape
    n_blocks = (size + _W - 1) // _W
    smooth = _SMOOTHING / (size - 2)
    c1 = (size - 2) * smooth * math.log(smooth) + _CONF * math.log(_CONF)
    t32 = target.astype(jnp.int32)
    g, x0 = _sc_gather(x, t32)
    out = pl.pallas_call(
        _make_tc_body(size, n_blocks, smooth, c1),
        grid=(n_blocks,),
        in_specs=[
            pl.BlockSpec((b, 1), lambda c: (0, 0)),
            pl.BlockSpec((b, 1), lambda c: (0, 0)),
            pl.BlockSpec((b, 1), lambda c: (0, 0)),
            pl.BlockSpec((b, _W), lambda c: (0, c)),
        ],
        out_specs=pl.BlockSpec((1, 1), lambda c: (0, 0), memory_space=pltpu.SMEM),
        out_shape=jax.ShapeDtypeStruct((1, 1), jnp.float32),
        scratch_shapes=[
            pltpu.VMEM((b, _L), jnp.float32),
        ],
        compiler_params=pltpu.CompilerParams(
            dimension_semantics=("arbitrary",),
        ),
    )(t32.reshape(b, 1), g.reshape(b, 1), x0.reshape(b, 1), x)
    return out[0, 0]
